# Initial kernel scaffold; baseline (speedup 1.0000x reference)
import jax
import jax.numpy as jnp
from jax.experimental import pallas as pl


def kernel(x_user, x_item, edge_index_u2i, edge_index_i2u, W_msg_u2i, W_self_u2i, W_msg_i2u, W_self_i2u):
    raise NotImplementedError("write your pallas kernel here")


# speedup vs baseline: 4.3797x; 4.3797x over previous
import functools

import jax
import jax.numpy as jnp
from jax import lax
from jax.experimental import pallas as pl
from jax.experimental.pallas import tpu as pltpu
from jax.experimental.pallas import tpu_sc as plsc

N = 10000
D = 128
E = 320000
NPAD = 10240
NC = 2
NS = 16
B = 128
G = 8
NG = 20
CH = G * NG
EPT = CH * B
E_PAD = NS * EPT
RPT = 624
TAIL = N - NS * RPT


def _tc_transform(x_user_p, x_item_p, W_msg_u2i, W_self_u2i, W_msg_i2u, W_self_i2u):
    BLK = 256

    def body(xu_ref, xi_ref, wm0_ref, ws0_ref, wm1_ref, ws1_ref, y_ref, b_ref):
        xu = xu_ref[...]
        xi = xi_ref[...]
        y_ref[0] = jnp.dot(xu, wm0_ref[...], preferred_element_type=jnp.float32)
        y_ref[1] = jnp.dot(xi, wm1_ref[...], preferred_element_type=jnp.float32)
        b_ref[0] = jnp.dot(xi, ws0_ref[...], preferred_element_type=jnp.float32)
        b_ref[1] = jnp.dot(xu, ws1_ref[...], preferred_element_type=jnp.float32)

    grid = (NPAD // BLK,)
    w_spec = pl.BlockSpec((D, D), lambda i: (0, 0))
    return pl.pallas_call(
        body,
        grid=grid,
        in_specs=[
            pl.BlockSpec((BLK, D), lambda i: (i, 0)),
            pl.BlockSpec((BLK, D), lambda i: (i, 0)),
            w_spec, w_spec, w_spec, w_spec,
        ],
        out_specs=[
            pl.BlockSpec((NC, BLK, D), lambda i: (0, i, 0)),
            pl.BlockSpec((NC, BLK, D), lambda i: (0, i, 0)),
        ],
        out_shape=[
            jax.ShapeDtypeStruct((NC, NPAD, D), jnp.float32),
            jax.ShapeDtypeStruct((NC, NPAD, D), jnp.float32),
        ],
    )(x_user_p, x_item_p, W_msg_u2i, W_self_u2i, W_msg_i2u, W_self_i2u)


def _sc_conv(y_flat, e_src, e_dst, base_all):
    mesh = plsc.VectorSubcoreMesh(core_axis_name="c", subcore_axis_name="s")

    @functools.partial(
        pl.kernel,
        out_type=(
            jax.ShapeDtypeStruct((N, D), jnp.float32),
            jax.ShapeDtypeStruct((N, D), jnp.float32),
        ),
        mesh=mesh,
        scratch_types=[
            pltpu.VMEM((G, B), jnp.int32),
            pltpu.VMEM((G, B), jnp.int32),
            pltpu.VMEM((B, D), jnp.float32),
            pltpu.VMEM((B, D), jnp.float32),
            pltpu.SemaphoreType.DMA,
            pltpu.SemaphoreType.DMA,
            pltpu.VMEM_SHARED((N, D), jnp.float32),
        ],
    )
    def k(y_ref, src_ref, dst_ref, base_ref, out_user, out_item,
          srcv, dstv, rows0, rows1, sem0, sem1, acc):
        cid = lax.axis_index("c")
        sid = lax.axis_index("s")
        row0 = pl.multiple_of(sid * RPT, 8)
        pltpu.sync_copy(base_ref.at[cid, pl.ds(row0, RPT)], acc.at[pl.ds(row0, RPT)])

        @pl.when(sid == NS - 1)
        def _():
            pltpu.sync_copy(base_ref.at[cid, pl.ds(NS * RPT, TAIL)],
                            acc.at[pl.ds(NS * RPT, TAIL)])

        plsc.subcore_barrier()

        bufs = ((rows0, sem0), (rows1, sem1))

        def group(g, carry):
            g0 = pl.multiple_of(g * G, 8)
            pltpu.sync_copy(src_ref.at[cid, sid, pl.ds(g0, G)], srcv)
            pltpu.sync_copy(dst_ref.at[cid, sid, pl.ds(g0, G)], dstv)
            pltpu.async_copy(y_ref.at[srcv.at[0]], rows0, sem0)
            pltpu.async_copy(y_ref.at[srcv.at[1]], rows1, sem1)
            for k in range(G):
                rows, sem = bufs[k % 2]
                pltpu.make_async_copy(y_ref.at[srcv.at[k]], rows, sem).wait()
                pltpu.sync_copy(rows, acc.at[dstv.at[k]], add=True)
                if k + 2 < G:
                    pltpu.async_copy(y_ref.at[srcv.at[k + 2]], rows, sem)
            return carry

        lax.fori_loop(0, NG, group, 0)
        plsc.subcore_barrier()

        @pl.when(cid == 0)
        def _():
            pltpu.sync_copy(acc.at[pl.ds(row0, RPT)], out_item.at[pl.ds(row0, RPT)])

            @pl.when(sid == NS - 1)
            def _():
                pltpu.sync_copy(acc.at[pl.ds(NS * RPT, TAIL)],
                                out_item.at[pl.ds(NS * RPT, TAIL)])

        @pl.when(cid == 1)
        def _():
            pltpu.sync_copy(acc.at[pl.ds(row0, RPT)], out_user.at[pl.ds(row0, RPT)])

            @pl.when(sid == NS - 1)
            def _():
                pltpu.sync_copy(acc.at[pl.ds(NS * RPT, TAIL)],
                                out_user.at[pl.ds(NS * RPT, TAIL)])

    return k(y_flat, e_src, e_dst, base_all)


def _prep_edges(edge_index_u2i, edge_index_i2u):
    src0 = edge_index_u2i[0].astype(jnp.int32)
    dst0 = edge_index_u2i[1].astype(jnp.int32)
    src1 = edge_index_i2u[0].astype(jnp.int32) + NPAD
    dst1 = edge_index_i2u[1].astype(jnp.int32)
    npad = E_PAD - E
    pad0 = jnp.full((npad,), N, jnp.int32)
    pad1 = jnp.full((npad,), NPAD + N, jnp.int32)
    padd = jnp.zeros((npad,), jnp.int32)
    e_src = jnp.stack([jnp.concatenate([src0, pad0]),
                       jnp.concatenate([src1, pad1])]).reshape(NC, NS, CH, B)
    e_dst = jnp.stack([jnp.concatenate([dst0, padd]),
                       jnp.concatenate([dst1, padd])]).reshape(NC, NS, CH, B)
    return e_src, e_dst


def kernel(x_user, x_item, edge_index_u2i, edge_index_i2u,
           W_msg_u2i, W_self_u2i, W_msg_i2u, W_self_i2u):
    x_user_p = jnp.pad(x_user, ((0, NPAD - N), (0, 0)))
    x_item_p = jnp.pad(x_item, ((0, NPAD - N), (0, 0)))
    e_src, e_dst = _prep_edges(edge_index_u2i, edge_index_i2u)
    y_all, base_all = _tc_transform(x_user_p, x_item_p,
                                    W_msg_u2i, W_self_u2i, W_msg_i2u, W_self_i2u)
    y_flat = y_all.reshape(NC * NPAD, D)
    out_user, out_item = _sc_conv(y_flat, e_src, e_dst, base_all)
    return (out_user, out_item)
